# baseline (speedup 1.0000x reference)
import functools

import jax
import jax.numpy as jnp
from jax import lax
from jax.experimental import pallas as pl
from jax.experimental.pallas import tpu as pltpu
from jax.experimental.pallas import tpu_sc as plsc

N = 10000
NP = 10240
E = 320000
D = 128
DH = D // 2
NC = 2
NS = 16
NW = NC * NS
WIN = 80
EPW = E // NW
NWIN_D = EPW // WIN
EPS = E // NS
NWIN_A = EPS // WIN
ROWS_PT = NP // NS
DEGW = (2 * NP) // NS

_MESH = plsc.VectorSubcoreMesh(core_axis_name="c", subcore_axis_name="s")


@functools.partial(
    pl.kernel,
    out_type=jax.ShapeDtypeStruct((NC, 2 * NP), jnp.float32),
    mesh=_MESH,
    scratch_types=[
        pltpu.VMEM((2 * NWIN_D, WIN), jnp.int32),
        pltpu.VMEM((WIN,), jnp.float32),
        pltpu.VMEM_SHARED((2 * NP,), jnp.float32),
        pltpu.SemaphoreType.DMA,
    ],
)
def _deg_kernel(idx_hbm, ones_hbm, zdeg_hbm, degp_hbm, idx_v, ones_v, deg_sh,
                dsem):
    cid = lax.axis_index("c")
    sid = lax.axis_index("s")
    tid = cid * NS + sid
    pltpu.sync_copy(zdeg_hbm, deg_sh.at[pl.ds(sid * DEGW, DEGW)])
    pltpu.sync_copy(idx_hbm.at[tid], idx_v)
    pltpu.sync_copy(ones_hbm, ones_v)
    plsc.subcore_barrier()

    def batch(bi, carry):
        for q in range(10):
            pltpu.async_copy(ones_v, deg_sh.at[idx_v.at[bi * 10 + q]], dsem,
                             add=True)
        for q in range(10):
            pltpu.make_async_copy(ones_v, deg_sh.at[idx_v.at[bi * 10 + q]],
                                  dsem).wait()
        return carry

    lax.fori_loop(0, (2 * NWIN_D) // 10, batch, 0)
    plsc.subcore_barrier()
    pltpu.sync_copy(deg_sh.at[pl.ds(sid * DEGW, DEGW)],
                    degp_hbm.at[cid, pl.ds(sid * DEGW, DEGW)])


@functools.partial(
    pl.kernel,
    out_type=jax.ShapeDtypeStruct((NC, NP, DH), jnp.float32),
    mesh=_MESH,
    scratch_types=[
        pltpu.VMEM((NWIN_A, WIN), jnp.int32),
        pltpu.VMEM((NWIN_A, WIN), jnp.int32),
        pltpu.VMEM((NWIN_A, WIN), jnp.float32),
        [pltpu.VMEM((WIN, DH), jnp.float32) for _ in range(5)],
        pltpu.VMEM_SHARED((NP, DH), jnp.float32),
        [pltpu.SemaphoreType.DMA for _ in range(5)],
        [pltpu.SemaphoreType.DMA for _ in range(5)],
    ],
    compiler_params=pltpu.CompilerParams(use_tc_tiling_on_sc=False),
)
def _agg_kernel(feat_hbm, src_hbm, dst_hbm, w_hbm, zrows_hbm, parts_hbm,
                src_v, dst_v, w_v, bufs, agg_sh, gsems, ssems):
    cid = lax.axis_index("c")
    sid = lax.axis_index("s")
    pltpu.sync_copy(zrows_hbm, agg_sh.at[pl.ds(sid * ROWS_PT, ROWS_PT)])
    pltpu.sync_copy(src_hbm.at[sid], src_v)
    pltpu.sync_copy(dst_hbm.at[sid], dst_v)
    pltpu.sync_copy(w_hbm.at[sid], w_v)

    coff = jnp.full((16,), cid * NP, dtype=jnp.int32)

    def fix(i, carry):
        j = i // (WIN // 16)
        g = i - j * (WIN // 16)
        sl = pl.ds(g * 16, 16)
        src_v[j, sl] = src_v[j, sl] + coff
        return carry

    lax.fori_loop(0, NWIN_A * (WIN // 16), fix, 0)
    plsc.subcore_barrier()

    def scale(buf, j):
        def group(g, inner):
            w16 = w_v[j, pl.ds(g * 16, 16)]
            for l in range(16):
                w = w16[l]
                for k in range(DH // 16):
                    sl = pl.ds(k * 16, 16)
                    buf[g * 16 + l, sl] = buf[g * 16 + l, sl] * w
            return inner

        lax.fori_loop(0, WIN // 16, group, 0)

    for q in range(3):
        pltpu.async_copy(feat_hbm.at[src_v.at[q]], bufs[q], gsems[q])

    def slot(q, base):
        j = base + q
        pltpu.make_async_copy(feat_hbm.at[src_v.at[j]], bufs[q],
                              gsems[q]).wait()
        scale(bufs[q], j)
        pltpu.async_copy(bufs[q], agg_sh.at[dst_v.at[j]], ssems[q], add=True)
        qn = (q + 3) % 5

        @pl.when(j >= 2)
        def _():
            pltpu.make_async_copy(bufs[qn], agg_sh.at[dst_v.at[j - 2]],
                                  ssems[qn]).wait()

        @pl.when(j + 3 < NWIN_A)
        def _():
            pltpu.async_copy(feat_hbm.at[src_v.at[j + 3]], bufs[qn],
                             gsems[qn])

    def block(i, carry):
        base = i * 5
        for q in range(5):
            slot(q, base)
        return carry

    lax.fori_loop(0, NWIN_A // 5, block, 0)
    pltpu.make_async_copy(bufs[3], agg_sh.at[dst_v.at[NWIN_A - 2]],
                          ssems[3]).wait()
    pltpu.make_async_copy(bufs[4], agg_sh.at[dst_v.at[NWIN_A - 1]],
                          ssems[4]).wait()
    plsc.subcore_barrier()
    pltpu.sync_copy(agg_sh.at[pl.ds(sid * ROWS_PT, ROWS_PT)],
                    parts_hbm.at[cid, pl.ds(sid * ROWS_PT, ROWS_PT)])


BLK = 1280


def _feat_body(x_ref, degp_ref, feat_ref):
    d = degp_ref[...]
    norm = lax.rsqrt(jnp.maximum(d[0] + d[1], 1.0))
    xb = x_ref[...] * norm[:, None]
    feat_ref[0] = xb[:, :DH]
    feat_ref[1] = xb[:, DH:]


_feat_call = pl.pallas_call(
    _feat_body,
    grid=(NP // BLK,),
    in_specs=[
        pl.BlockSpec((BLK, D), lambda i: (i, 0)),
        pl.BlockSpec((2, BLK), lambda i: (0, i)),
    ],
    out_specs=pl.BlockSpec((NC, BLK, DH), lambda i: (0, i, 0)),
    out_shape=jax.ShapeDtypeStruct((NC, NP, DH), jnp.float32),
)


def _out_body(p_ref, w_ref, degp_ref, b_ref, o_ref):
    p = p_ref[...]
    a = jnp.concatenate([p[0], p[1]], axis=1)
    r = jnp.dot(a, w_ref[...], preferred_element_type=jnp.float32)
    d = degp_ref[...]
    norm = lax.rsqrt(jnp.maximum(d[0] + d[1], 1.0))
    o_ref[...] = r * norm[:, None] + b_ref[...]


_out_call = pl.pallas_call(
    _out_body,
    grid=(NP // BLK,),
    in_specs=[
        pl.BlockSpec((NC, BLK, DH), lambda i: (0, i, 0)),
        pl.BlockSpec((D, D), lambda i: (0, 0)),
        pl.BlockSpec((2, BLK), lambda i: (0, NP // BLK + i)),
        pl.BlockSpec((1, D), lambda i: (0, 0)),
    ],
    out_specs=pl.BlockSpec((BLK, D), lambda i: (i, 0)),
    out_shape=jax.ShapeDtypeStruct((NP, D), jnp.float32),
)


def kernel(node_embedding, edge_embedding, edge_index, W, b):
    ei = edge_index.astype(jnp.int32)
    src_d = ei[0].reshape(NW, NWIN_D, WIN)
    dst_d = ei[1].reshape(NW, NWIN_D, WIN)
    deg_idx = jnp.concatenate([src_d, dst_d + NP], axis=1)
    src_a = ei[0].reshape(NS, NWIN_A, WIN)
    dst_a = ei[1].reshape(NS, NWIN_A, WIN)
    wts = edge_embedding.astype(jnp.float32).reshape(NS, NWIN_A, WIN)
    ones = jnp.ones((WIN,), jnp.float32)
    zdeg = jnp.zeros((DEGW,), jnp.float32)
    zrows = jnp.zeros((ROWS_PT, DH), jnp.float32)
    x_pad = jnp.zeros((NP, D), jnp.float32).at[:N].set(node_embedding)

    degp = _deg_kernel(deg_idx, ones, zdeg)
    feat = _feat_call(x_pad, degp).reshape(2 * NP, DH)
    parts = _agg_kernel(feat, src_a, dst_a, wts, zrows)
    return _out_call(parts, W, degp, b.reshape(1, D))[:N]


# speedup vs baseline: 1.7046x; 1.7046x over previous
import functools

import jax
import jax.numpy as jnp
from jax import lax
from jax.experimental import pallas as pl
from jax.experimental.pallas import tpu as pltpu
from jax.experimental.pallas import tpu_sc as plsc

N = 10000
NP = 10240
E = 320000
D = 128
DH = D // 2
NC = 2
NS = 16
NW = NC * NS
WIN = 80
EPW = E // NW
NWIN_D = EPW // WIN
EPS = E // NS
NWIN_A = EPS // WIN
ROWS_PT = NP // NS
DEGW = (2 * NP) // NS

_MESH = plsc.VectorSubcoreMesh(core_axis_name="c", subcore_axis_name="s")


@functools.partial(
    pl.kernel,
    out_type=jax.ShapeDtypeStruct((NC, 2 * NP), jnp.float32),
    mesh=_MESH,
    scratch_types=[
        pltpu.VMEM((2 * NWIN_D, WIN), jnp.int32),
        pltpu.VMEM((WIN,), jnp.float32),
        pltpu.VMEM_SHARED((2 * NP,), jnp.float32),
        pltpu.SemaphoreType.DMA,
    ],
)
def _deg_kernel(idx_hbm, ones_hbm, zdeg_hbm, degp_hbm, idx_v, ones_v, deg_sh,
                dsem):
    cid = lax.axis_index("c")
    sid = lax.axis_index("s")
    tid = cid * NS + sid
    pltpu.sync_copy(zdeg_hbm, deg_sh.at[pl.ds(sid * DEGW, DEGW)])
    pltpu.sync_copy(idx_hbm.at[tid], idx_v)
    pltpu.sync_copy(ones_hbm, ones_v)
    plsc.subcore_barrier()

    def batch(bi, carry):
        for q in range(10):
            pltpu.async_copy(ones_v, deg_sh.at[idx_v.at[bi * 10 + q]], dsem,
                             add=True)
        for q in range(10):
            pltpu.make_async_copy(ones_v, deg_sh.at[idx_v.at[bi * 10 + q]],
                                  dsem).wait()
        return carry

    lax.fori_loop(0, (2 * NWIN_D) // 10, batch, 0)
    plsc.subcore_barrier()
    pltpu.sync_copy(deg_sh.at[pl.ds(sid * DEGW, DEGW)],
                    degp_hbm.at[cid, pl.ds(sid * DEGW, DEGW)])


@functools.partial(
    pl.kernel,
    out_type=jax.ShapeDtypeStruct((NC, NP, DH), jnp.float32),
    mesh=_MESH,
    scratch_types=[
        pltpu.VMEM((NWIN_A, WIN), jnp.int32),
        pltpu.VMEM((NWIN_A, WIN), jnp.int32),
        pltpu.VMEM((NWIN_A, WIN), jnp.float32),
        [pltpu.VMEM((WIN, DH), jnp.float32) for _ in range(5)],
        pltpu.VMEM_SHARED((NP, DH), jnp.float32),
        [pltpu.SemaphoreType.DMA for _ in range(5)],
        [pltpu.SemaphoreType.DMA for _ in range(5)],
    ],
    compiler_params=pltpu.CompilerParams(use_tc_tiling_on_sc=False),
)
def _agg_kernel(feat_hbm, src_hbm, dst_hbm, w_hbm, zrows_hbm, parts_hbm,
                src_v, dst_v, w_v, bufs, agg_sh, gsems, ssems):
    cid = lax.axis_index("c")
    sid = lax.axis_index("s")
    pltpu.sync_copy(zrows_hbm, agg_sh.at[pl.ds(sid * ROWS_PT, ROWS_PT)])
    pltpu.sync_copy(src_hbm.at[sid], src_v)
    pltpu.sync_copy(dst_hbm.at[sid], dst_v)
    pltpu.sync_copy(w_hbm.at[sid], w_v)

    coff = jnp.full((16,), cid * NP, dtype=jnp.int32)

    def fix(i, carry):
        j = i // (WIN // 16)
        g = i - j * (WIN // 16)
        sl = pl.ds(g * 16, 16)
        src_v[j, sl] = src_v[j, sl] + coff
        return carry

    lax.fori_loop(0, NWIN_A * (WIN // 16), fix, 0)
    plsc.subcore_barrier()

    def scale(buf, j):
        for g in range(WIN // 16):
            w16 = w_v[j, pl.ds(g * 16, 16)]
            for l in range(16):
                w = w16[l]
                for k in range(DH // 16):
                    sl = pl.ds(k * 16, 16)
                    buf[g * 16 + l, sl] = buf[g * 16 + l, sl] * w

    for q in range(3):
        pltpu.async_copy(feat_hbm.at[src_v.at[q]], bufs[q], gsems[q])

    def slot(q, base):
        j = base + q
        pltpu.make_async_copy(feat_hbm.at[src_v.at[j]], bufs[q],
                              gsems[q]).wait()
        scale(bufs[q], j)
        pltpu.async_copy(bufs[q], agg_sh.at[dst_v.at[j]], ssems[q], add=True)
        qn = (q + 3) % 5

        @pl.when(j >= 2)
        def _():
            pltpu.make_async_copy(bufs[qn], agg_sh.at[dst_v.at[j - 2]],
                                  ssems[qn]).wait()

        @pl.when(j + 3 < NWIN_A)
        def _():
            pltpu.async_copy(feat_hbm.at[src_v.at[j + 3]], bufs[qn],
                             gsems[qn])

    def block(i, carry):
        base = i * 5
        for q in range(5):
            slot(q, base)
        return carry

    lax.fori_loop(0, NWIN_A // 5, block, 0)
    pltpu.make_async_copy(bufs[3], agg_sh.at[dst_v.at[NWIN_A - 2]],
                          ssems[3]).wait()
    pltpu.make_async_copy(bufs[4], agg_sh.at[dst_v.at[NWIN_A - 1]],
                          ssems[4]).wait()
    plsc.subcore_barrier()
    pltpu.sync_copy(agg_sh.at[pl.ds(sid * ROWS_PT, ROWS_PT)],
                    parts_hbm.at[cid, pl.ds(sid * ROWS_PT, ROWS_PT)])


BLK = 1280


def _feat_body(x_ref, degp_ref, feat_ref):
    d = degp_ref[...]
    norm = lax.rsqrt(jnp.maximum(d[0] + d[1], 1.0))
    xb = x_ref[...] * norm[:, None]
    feat_ref[0] = xb[:, :DH]
    feat_ref[1] = xb[:, DH:]


_feat_call = pl.pallas_call(
    _feat_body,
    grid=(NP // BLK,),
    in_specs=[
        pl.BlockSpec((BLK, D), lambda i: (i, 0)),
        pl.BlockSpec((2, BLK), lambda i: (0, i)),
    ],
    out_specs=pl.BlockSpec((NC, BLK, DH), lambda i: (0, i, 0)),
    out_shape=jax.ShapeDtypeStruct((NC, NP, DH), jnp.float32),
)


def _out_body(p_ref, w_ref, degp_ref, b_ref, o_ref):
    p = p_ref[...]
    a = jnp.concatenate([p[0], p[1]], axis=1)
    r = jnp.dot(a, w_ref[...], preferred_element_type=jnp.float32)
    d = degp_ref[...]
    norm = lax.rsqrt(jnp.maximum(d[0] + d[1], 1.0))
    o_ref[...] = r * norm[:, None] + b_ref[...]


_out_call = pl.pallas_call(
    _out_body,
    grid=(NP // BLK,),
    in_specs=[
        pl.BlockSpec((NC, BLK, DH), lambda i: (0, i, 0)),
        pl.BlockSpec((D, D), lambda i: (0, 0)),
        pl.BlockSpec((2, BLK), lambda i: (0, NP // BLK + i)),
        pl.BlockSpec((1, D), lambda i: (0, 0)),
    ],
    out_specs=pl.BlockSpec((BLK, D), lambda i: (i, 0)),
    out_shape=jax.ShapeDtypeStruct((NP, D), jnp.float32),
)


def kernel(node_embedding, edge_embedding, edge_index, W, b):
    ei = edge_index.astype(jnp.int32)
    src_d = ei[0].reshape(NW, NWIN_D, WIN)
    dst_d = ei[1].reshape(NW, NWIN_D, WIN)
    deg_idx = jnp.concatenate([src_d, dst_d + NP], axis=1)
    src_a = ei[0].reshape(NS, NWIN_A, WIN)
    dst_a = ei[1].reshape(NS, NWIN_A, WIN)
    wts = edge_embedding.astype(jnp.float32).reshape(NS, NWIN_A, WIN)
    ones = jnp.ones((WIN,), jnp.float32)
    zdeg = jnp.zeros((DEGW,), jnp.float32)
    zrows = jnp.zeros((ROWS_PT, DH), jnp.float32)
    x_pad = jnp.zeros((NP, D), jnp.float32).at[:N].set(node_embedding)

    degp = _deg_kernel(deg_idx, ones, zdeg)
    feat = _feat_call(x_pad, degp).reshape(2 * NP, DH)
    parts = _agg_kernel(feat, src_a, dst_a, wts, zrows)
    return _out_call(parts, W, degp, b.reshape(1, D))[:N]
